# baseline (speedup 1.0000x reference)
import functools

import jax
import jax.numpy as jnp
from jax import lax
from jax.experimental import pallas as pl
from jax.experimental.pallas import tpu as pltpu
from jax.experimental.pallas import tpu_sc as plsc

VOCAB = 1000000
HIDDEN = 64
BATCH = 16384
HIST = 50
B_TOTAL = BATCH * HIST

NUM_CORES = 2
NUM_SUBCORES = 16
NW = NUM_CORES * NUM_SUBCORES
BLK = 128
N_UNITS = B_TOTAL // BLK
U_PER_W = N_UNITS // NW
NBUF = 4
N_GROUPS = U_PER_W // NBUF
HT = HIDDEN // 8
BC = BATCH // BLK


def _make_gather():
    mesh = plsc.VectorSubcoreMesh(core_axis_name="c", subcore_axis_name="s")

    @functools.partial(
        pl.kernel,
        mesh=mesh,
        out_type=jax.ShapeDtypeStruct((HIST, HT, BC, 8, BLK), jnp.float32),
        scratch_types=[
            pltpu.VMEM((U_PER_W * BLK,), jnp.int32),
            *[pltpu.VMEM((BLK, HIDDEN), jnp.float32) for _ in range(NBUF)],
            *[pltpu.VMEM((HT, 8, BLK), jnp.float32) for _ in range(NBUF)],
            *[pltpu.SemaphoreType.DMA for _ in range(2 * NBUF)],
        ],
        compiler_params=pltpu.CompilerParams(
            use_tc_tiling_on_sc=False, needs_layout_passes=False
        ),
    )
    def gather_kernel(idx_hbm, table_hbm, out_hbm, idx_v, *bufs):
        rows = bufs[:NBUF]
        outt = bufs[NBUF : 2 * NBUF]
        sg = bufs[2 * NBUF : 3 * NBUF]
        ss = bufs[3 * NBUF : 4 * NBUF]
        wid = lax.axis_index("s") * NUM_CORES + lax.axis_index("c")
        u0 = wid * U_PER_W

        pltpu.sync_copy(idx_hbm.at[pl.ds(u0 * BLK, U_PER_W * BLK)], idx_v)

        lane = lax.iota(jnp.int32, 16)
        row_ids = [lane + 16 * k for k in range(8)]

        def g_desc(ul, b):
            src = table_hbm.at[idx_v.at[pl.ds(ul * BLK, BLK)]]
            return pltpu.make_async_copy(src, rows[b], sg[b])

        def s_desc(ul, b):
            ug = u0 + ul
            j = ug // BC
            bc = ug % BC
            return pltpu.make_async_copy(outt[b], out_hbm.at[j, :, bc], ss[b])

        def transpose(b):
            for tr in range(HT):

                def hbody(h8, _, tr=tr):
                    hv = jnp.full((16,), tr * 8 + h8, jnp.int32)
                    for k in range(8):
                        vec = plsc.load_gather(rows[b], [row_ids[k], hv])
                        outt[b][tr, h8, pl.ds(k * 16, 16)] = vec
                    return 0

                lax.fori_loop(0, 8, hbody, 0)

        for b in range(NBUF):
            g_desc(b, b).start()
        for b in range(NBUF):
            g_desc(b, b).wait()
            transpose(b)
            s_desc(b, b).start()
            g_desc(b + NBUF, b).start()

        def group(g, _):
            ub = g * NBUF
            for b in range(NBUF):
                ul = ub + b
                s_desc(ul - NBUF, b).wait()
                g_desc(ul, b).wait()
                transpose(b)
                s_desc(ul, b).start()
                g_desc(ul + NBUF, b).start()
            return 0

        lax.fori_loop(1, N_GROUPS - 1, group, 0)

        ub = (N_GROUPS - 1) * NBUF
        for b in range(NBUF):
            ul = ub + b
            s_desc(ul - NBUF, b).wait()
            g_desc(ul, b).wait()
            transpose(b)
            s_desc(ul, b).start()
        for b in range(NBUF):
            s_desc(ub + b, b).wait()

    return gather_kernel


_gather = _make_gather()


@jax.jit
def kernel(tokens, embedding):
    idx = tokens.astype(jnp.int32).T.reshape(B_TOTAL)
    out5 = _gather(idx, embedding)
    return out5.transpose(2, 4, 0, 1, 3).reshape(BATCH, HIST, HIDDEN)


# speedup vs baseline: 1.5348x; 1.5348x over previous
import functools

import jax
import jax.numpy as jnp
from jax import lax
from jax.experimental import pallas as pl
from jax.experimental.pallas import tpu as pltpu
from jax.experimental.pallas import tpu_sc as plsc

VOCAB = 1000000
HIDDEN = 64
BATCH = 16384
HIST = 50
B_TOTAL = BATCH * HIST

NUM_CORES = 2
NUM_SUBCORES = 16
NW = NUM_CORES * NUM_SUBCORES
BLK = 128
N_UNITS = B_TOTAL // BLK
U_PER_W = N_UNITS // NW
NBUF = 4
N_GROUPS = U_PER_W // NBUF
HT = HIDDEN // 8
BC = BATCH // BLK


def _make_gather():
    mesh = plsc.VectorSubcoreMesh(core_axis_name="c", subcore_axis_name="s")

    @functools.partial(
        pl.kernel,
        mesh=mesh,
        out_type=jax.ShapeDtypeStruct((HIST, HT, BC, 8, BLK), jnp.float32),
        scratch_types=[
            pltpu.VMEM((U_PER_W * BLK,), jnp.int32),
            *[pltpu.VMEM((BLK, HIDDEN), jnp.float32) for _ in range(NBUF)],
            *[pltpu.VMEM((HT, 8, BLK), jnp.float32) for _ in range(NBUF)],
            *[pltpu.SemaphoreType.DMA for _ in range(2 * NBUF)],
        ],
        compiler_params=pltpu.CompilerParams(
            use_tc_tiling_on_sc=False, needs_layout_passes=False
        ),
    )
    def gather_kernel(idx_hbm, table_hbm, out_hbm, idx_v, *bufs):
        rows = bufs[:NBUF]
        outt = bufs[NBUF : 2 * NBUF]
        sg = bufs[2 * NBUF : 3 * NBUF]
        ss = bufs[3 * NBUF : 4 * NBUF]
        wid = lax.axis_index("s") * NUM_CORES + lax.axis_index("c")
        u0 = wid * U_PER_W

        pltpu.sync_copy(idx_hbm.at[pl.ds(u0 * BLK, U_PER_W * BLK)], idx_v)

        lane = lax.iota(jnp.int32, 16)
        row_k = [lane + 16 * k for k in range(8)]

        def g_desc(ul, b):
            src = table_hbm.at[idx_v.at[pl.ds(ul * BLK, BLK)]]
            return pltpu.make_async_copy(src, rows[b], sg[b])

        def s_desc(ul, b):
            ug = u0 + ul
            j = ug // BC
            bc = ug % BC
            return pltpu.make_async_copy(outt[b], out_hbm.at[j, :, bc], ss[b])

        def transpose(b):
            def h0body(h0, _):
                hh = (lane + h0) & 63
                trv = hh >> 3
                h8v = hh & 7
                for k in range(8):
                    vec = plsc.load_gather(rows[b], [row_k[k], hh])
                    plsc.store_scatter(outt[b], [trv, h8v, row_k[k]], vec)
                return 0

            lax.fori_loop(0, HIDDEN, h0body, 0)

        for b in range(NBUF):
            g_desc(b, b).start()

        def group(g, _):
            for b in range(NBUF):
                ul = g * NBUF + b

                @pl.when(g > 0)
                def _(ul=ul, b=b):
                    s_desc(ul - NBUF, b).wait()

                g_desc(ul, b).wait()
                transpose(b)
                s_desc(ul, b).start()

                @pl.when(g < N_GROUPS - 1)
                def _(ul=ul, b=b):
                    g_desc(ul + NBUF, b).start()

            return 0

        lax.fori_loop(0, N_GROUPS, group, 0)

        for b in range(NBUF):
            s_desc((N_GROUPS - 1) * NBUF + b, b).wait()

    return gather_kernel


_gather = _make_gather()


@jax.jit
def kernel(tokens, embedding):
    idx = tokens.astype(jnp.int32).T.reshape(B_TOTAL)
    out5 = _gather(idx, embedding)
    return out5.transpose(2, 4, 0, 1, 3).reshape(BATCH, HIST, HIDDEN)


# speedup vs baseline: 2.0825x; 1.3569x over previous
import functools

import jax
import jax.numpy as jnp
from jax import lax
from jax.experimental import pallas as pl
from jax.experimental.pallas import tpu as pltpu
from jax.experimental.pallas import tpu_sc as plsc

VOCAB = 1000000
HIDDEN = 64
BATCH = 16384
HIST = 50
B_TOTAL = BATCH * HIST

NUM_CORES = 2
NUM_SUBCORES = 16
NW = NUM_CORES * NUM_SUBCORES
BLK = 128
N_UNITS = B_TOTAL // BLK
U_PER_W = N_UNITS // NW
NBUF = 4
N_GROUPS = U_PER_W // NBUF
HT = HIDDEN // 8
BC = BATCH // BLK


def _make_gather():
    mesh = plsc.VectorSubcoreMesh(core_axis_name="c", subcore_axis_name="s")

    @functools.partial(
        pl.kernel,
        mesh=mesh,
        out_type=jax.ShapeDtypeStruct((HIST, HT, BC, 8, BLK), jnp.float32),
        scratch_types=[
            pltpu.VMEM((U_PER_W * BLK,), jnp.int32),
            *[pltpu.VMEM((BLK, HIDDEN), jnp.float32) for _ in range(NBUF)],
            *[pltpu.VMEM((HT, 8, BLK), jnp.float32) for _ in range(NBUF)],
            *[pltpu.SemaphoreType.DMA for _ in range(2 * NBUF)],
        ],
        compiler_params=pltpu.CompilerParams(
            use_tc_tiling_on_sc=False, needs_layout_passes=False
        ),
    )
    def gather_kernel(idx_hbm, table_hbm, out_hbm, idx_v, *bufs):
        rows = bufs[:NBUF]
        outt = bufs[NBUF : 2 * NBUF]
        sg = bufs[2 * NBUF : 3 * NBUF]
        ss = bufs[3 * NBUF : 4 * NBUF]
        wid = lax.axis_index("s") * NUM_CORES + lax.axis_index("c")
        u0 = wid * U_PER_W

        pltpu.sync_copy(idx_hbm.at[pl.ds(u0 * BLK, U_PER_W * BLK)], idx_v)

        lane = lax.iota(jnp.int32, 16)
        row_k = [lane + 16 * k for k in range(8)]

        def g_desc(ul, b):
            src = table_hbm.at[idx_v.at[pl.ds(ul * BLK, BLK)]]
            return pltpu.make_async_copy(src, rows[b], sg[b])

        def s_desc(ul, b):
            ug = u0 + ul
            j = ug // BC
            bc = ug % BC
            return pltpu.make_async_copy(outt[b], out_hbm.at[j, :, bc], ss[b])

        def transpose(b):
            @plsc.parallel_loop(0, HIDDEN, step=1, unroll=4)
            def h0body(h0):
                hh = (lane + h0) & 63
                trv = hh >> 3
                h8v = hh & 7
                for k in range(8):
                    vec = plsc.load_gather(rows[b], [row_k[k], hh])
                    plsc.store_scatter(outt[b], [trv, h8v, row_k[k]], vec)

        for b in range(NBUF):
            g_desc(b, b).start()

        def group(g, _):
            for b in range(NBUF):
                ul = g * NBUF + b

                @pl.when(g > 0)
                def _(ul=ul, b=b):
                    s_desc(ul - NBUF, b).wait()

                g_desc(ul, b).wait()
                transpose(b)
                s_desc(ul, b).start()

                @pl.when(g < N_GROUPS - 1)
                def _(ul=ul, b=b):
                    g_desc(ul + NBUF, b).start()

            return 0

        lax.fori_loop(0, N_GROUPS, group, 0)

        for b in range(NBUF):
            s_desc((N_GROUPS - 1) * NBUF + b, b).wait()

    return gather_kernel


_gather = _make_gather()


@jax.jit
def kernel(tokens, embedding):
    idx = tokens.astype(jnp.int32).T.reshape(B_TOTAL)
    out5 = _gather(idx, embedding)
    return out5.transpose(2, 4, 0, 1, 3).reshape(BATCH, HIST, HIDDEN)


# speedup vs baseline: 3.7731x; 1.8118x over previous
import functools

import jax
import jax.numpy as jnp
from jax import lax
from jax.experimental import pallas as pl
from jax.experimental.pallas import tpu as pltpu
from jax.experimental.pallas import tpu_sc as plsc

VOCAB = 1000000
HIDDEN = 64
BATCH = 16384
HIST = 50
B_TOTAL = BATCH * HIST

NUM_CORES = 2
NUM_SUBCORES = 16
NW = NUM_CORES * NUM_SUBCORES
PACKED = 2 * HIDDEN

VPAD = 1000064
NBLK = VPAD // 128
A_PER_W = -(-NBLK // NW)
A_NBUF = 5
A_GROUPS = A_PER_W // A_NBUF
PROWS = VPAD // 2

BLK = 128
N_UNITS = B_TOTAL // BLK
U_PER_W = N_UNITS // NW
B_NBUF = 4
B_GROUPS = U_PER_W // B_NBUF
HT = HIDDEN // 8
BC = BATCH // BLK

_params = pltpu.CompilerParams(use_tc_tiling_on_sc=True, needs_layout_passes=False)


def _make_format():
    mesh = plsc.VectorSubcoreMesh(core_axis_name="c", subcore_axis_name="s")

    @functools.partial(
        pl.kernel,
        mesh=mesh,
        out_type=jax.ShapeDtypeStruct((PROWS, PACKED), jnp.float32),
        scratch_types=[
            *[pltpu.VMEM((8, 8, 128), jnp.float32) for _ in range(A_NBUF)],
            *[pltpu.VMEM((HIDDEN, PACKED), jnp.float32) for _ in range(A_NBUF)],
            *[pltpu.SemaphoreType.DMA for _ in range(2 * A_NBUF)],
        ],
        compiler_params=_params,
    )
    def format_kernel(embt_hbm, out_hbm, *bufs):
        tin = bufs[:A_NBUF]
        tout = bufs[A_NBUF : 2 * A_NBUF]
        sg = bufs[2 * A_NBUF : 3 * A_NBUF]
        ss = bufs[3 * A_NBUF : 4 * A_NBUF]
        wid = lax.axis_index("s") * NUM_CORES + lax.axis_index("c")
        c0 = wid * A_PER_W

        lane = lax.iota(jnp.int32, 16)
        vv = [lane + 16 * m for m in range(8)]
        pv = [v >> 1 for v in vv]
        sv = [(v & 1) << 6 for v in vv]

        def g_descs(c, b):
            return [
                pltpu.make_async_copy(
                    embt_hbm.at[pl.ds(tr * 8, 8), pl.ds(c * 128, 128)],
                    tin[b].at[tr],
                    sg[b],
                )
                for tr in range(8)
            ]

        def s_desc(c, b):
            return pltpu.make_async_copy(
                tout[b], out_hbm.at[pl.ds(c * 64, 64)], ss[b]
            )

        def transpose(b):
            @plsc.parallel_loop(0, HIDDEN, step=1, unroll=4)
            def h0body(h0):
                hh = (lane + h0) & 63
                trv = hh >> 3
                h8v = hh & 7
                for m in range(8):
                    vec = plsc.load_gather(tin[b], [trv, h8v, vv[m]])
                    plsc.store_scatter(tout[b], [pv[m], sv[m] | hh], vec)

        def start_block(c, b):
            @pl.when(c < NBLK)
            def _():
                for d in g_descs(c, b):
                    d.start()

        def finish_block(c, b):
            @pl.when(c < NBLK)
            def _():
                for d in g_descs(c, b):
                    d.wait()
                transpose(b)
                s_desc(c, b).start()

        for b in range(A_NBUF):
            start_block(c0 + b, b)

        def group(g, _):
            for b in range(A_NBUF):
                cl = g * A_NBUF + b
                c = c0 + cl

                @pl.when((g > 0) & (c - A_NBUF < NBLK))
                def _(c=c, b=b):
                    s_desc(c - A_NBUF, b).wait()

                finish_block(c, b)

                @pl.when(g < A_GROUPS - 1)
                def _(c=c, b=b):
                    start_block(c + A_NBUF, b)

            return 0

        lax.fori_loop(0, A_GROUPS, group, 0)

        for b in range(A_NBUF):
            c = c0 + A_PER_W - A_NBUF + b

            @pl.when(c < NBLK)
            def _(c=c, b=b):
                s_desc(c, b).wait()

    return format_kernel


def _make_lookup():
    mesh = plsc.VectorSubcoreMesh(core_axis_name="c", subcore_axis_name="s")

    @functools.partial(
        pl.kernel,
        mesh=mesh,
        out_type=jax.ShapeDtypeStruct((HIST, HT, BC, 8, BLK), jnp.float32),
        scratch_types=[
            pltpu.VMEM((U_PER_W * BLK,), jnp.int32),
            *[pltpu.VMEM((BLK,), jnp.int32) for _ in range(B_NBUF)],
            *[pltpu.VMEM((BLK, PACKED), jnp.float32) for _ in range(B_NBUF)],
            *[pltpu.VMEM((HT, 8, BLK), jnp.float32) for _ in range(B_NBUF)],
            *[pltpu.SemaphoreType.DMA for _ in range(2 * B_NBUF)],
        ],
        compiler_params=_params,
    )
    def lookup_kernel(idx_hbm, table_hbm, out_hbm, idx_v, *bufs):
        idxp = bufs[:B_NBUF]
        rows = bufs[B_NBUF : 2 * B_NBUF]
        outt = bufs[2 * B_NBUF : 3 * B_NBUF]
        sg = bufs[3 * B_NBUF : 4 * B_NBUF]
        ss = bufs[4 * B_NBUF : 5 * B_NBUF]
        wid = lax.axis_index("s") * NUM_CORES + lax.axis_index("c")
        u0 = wid * U_PER_W

        pltpu.sync_copy(idx_hbm.at[pl.ds(u0 * BLK, U_PER_W * BLK)], idx_v)

        lane = lax.iota(jnp.int32, 16)
        row_k = [lane + 16 * k for k in range(8)]

        def fill_idxp(ul, b):
            for k in range(8):
                v = idx_v[pl.ds(ul * BLK + 16 * k, 16)]
                idxp[b][pl.ds(16 * k, 16)] = v >> 1

        def g_desc(ul, b):
            src = table_hbm.at[idxp[b]]
            return pltpu.make_async_copy(src, rows[b], sg[b])

        def s_desc(ul, b):
            ug = u0 + ul
            j = ug // BC
            bc = ug % BC
            return pltpu.make_async_copy(outt[b], out_hbm.at[j, :, bc], ss[b])

        def transpose(ul, b):
            par = [
                ((idx_v[pl.ds(ul * BLK + 16 * k, 16)] & 1) << 6) for k in range(8)
            ]

            @plsc.parallel_loop(0, HIDDEN, step=1, unroll=4)
            def h0body(h0):
                hh = (lane + h0) & 63
                trv = hh >> 3
                h8v = hh & 7
                for k in range(8):
                    vec = plsc.load_gather(rows[b], [row_k[k], hh | par[k]])
                    plsc.store_scatter(outt[b], [trv, h8v, row_k[k]], vec)

        for b in range(B_NBUF):
            fill_idxp(b, b)
            g_desc(b, b).start()

        def group(g, _):
            for b in range(B_NBUF):
                ul = g * B_NBUF + b

                @pl.when(g > 0)
                def _(ul=ul, b=b):
                    s_desc(ul - B_NBUF, b).wait()

                g_desc(ul, b).wait()
                transpose(ul, b)
                s_desc(ul, b).start()

                @pl.when(g < B_GROUPS - 1)
                def _(ul=ul, b=b):
                    fill_idxp(ul + B_NBUF, b)
                    g_desc(ul + B_NBUF, b).start()

            return 0

        lax.fori_loop(0, B_GROUPS, group, 0)

        for b in range(B_NBUF):
            s_desc(U_PER_W - B_NBUF + b, b).wait()

    return lookup_kernel


_format = _make_format()
_lookup = _make_lookup()


@jax.jit
def kernel(tokens, embedding):
    idx = tokens.astype(jnp.int32).T.reshape(B_TOTAL)
    table = _format(embedding.T)
    out5 = _lookup(idx, table)
    return out5.transpose(2, 4, 0, 1, 3).reshape(BATCH, HIST, HIDDEN)


# speedup vs baseline: 4.5984x; 1.2187x over previous
import functools

import jax
import jax.numpy as jnp
from jax import lax
from jax.experimental import pallas as pl
from jax.experimental.pallas import tpu as pltpu
from jax.experimental.pallas import tpu_sc as plsc

VOCAB = 1000000
HIDDEN = 64
BATCH = 16384
HIST = 50
B_TOTAL = BATCH * HIST

NUM_CORES = 2
NUM_SUBCORES = 16
NW = NUM_CORES * NUM_SUBCORES
PACKED = 2 * HIDDEN

VPAD = 1000064
NBLK = VPAD // 128
A_PER_W = -(-NBLK // NW)
A_NBUF = 5
A_GROUPS = A_PER_W // A_NBUF
PROWS = VPAD // 2

BLK = 128
N_UNITS = B_TOTAL // BLK
U_PER_W = N_UNITS // NW
B_NBUF = 4
B_GROUPS = U_PER_W // B_NBUF
HT = HIDDEN // 8
BC = BATCH // BLK

_params = pltpu.CompilerParams(use_tc_tiling_on_sc=True, needs_layout_passes=False)


def _make_format():
    mesh = plsc.VectorSubcoreMesh(core_axis_name="c", subcore_axis_name="s")

    @functools.partial(
        pl.kernel,
        mesh=mesh,
        out_type=jax.ShapeDtypeStruct((PROWS, PACKED), jnp.float32),
        scratch_types=[
            *[pltpu.VMEM((8, 8, 128), jnp.float32) for _ in range(A_NBUF)],
            *[pltpu.VMEM((HIDDEN, PACKED), jnp.float32) for _ in range(A_NBUF)],
            *[pltpu.SemaphoreType.DMA for _ in range(2 * A_NBUF)],
        ],
        compiler_params=_params,
    )
    def format_kernel(embt_hbm, out_hbm, *bufs):
        tin = bufs[:A_NBUF]
        tout = bufs[A_NBUF : 2 * A_NBUF]
        sg = bufs[2 * A_NBUF : 3 * A_NBUF]
        ss = bufs[3 * A_NBUF : 4 * A_NBUF]
        wid = lax.axis_index("s") * NUM_CORES + lax.axis_index("c")
        c0 = wid * A_PER_W

        lane = lax.iota(jnp.int32, 16)
        vv = [lane + 16 * m for m in range(8)]
        pv = [v >> 1 for v in vv]
        sv = [(v & 1) << 6 for v in vv]

        def g_descs(c, b):
            return [
                pltpu.make_async_copy(
                    embt_hbm.at[pl.ds(tr * 8, 8), pl.ds(c * 128, 128)],
                    tin[b].at[tr],
                    sg[b],
                )
                for tr in range(8)
            ]

        def s_desc(c, b):
            return pltpu.make_async_copy(
                tout[b], out_hbm.at[pl.ds(c * 64, 64)], ss[b]
            )

        def transpose(b):
            @plsc.parallel_loop(0, HIDDEN, step=1, unroll=4)
            def h0body(h0):
                hh = (lane + h0) & 63
                trv = hh >> 3
                h8v = hh & 7
                for m in range(8):
                    vec = plsc.load_gather(tin[b], [trv, h8v, vv[m]])
                    plsc.store_scatter(tout[b], [pv[m], sv[m] | hh], vec)

        def start_block(c, b):
            @pl.when(c < NBLK)
            def _():
                for d in g_descs(c, b):
                    d.start()

        def finish_block(c, b):
            @pl.when(c < NBLK)
            def _():
                for d in g_descs(c, b):
                    d.wait()
                transpose(b)
                s_desc(c, b).start()

        for b in range(A_NBUF):
            start_block(c0 + b, b)

        def group(g, _):
            for b in range(A_NBUF):
                cl = g * A_NBUF + b
                c = c0 + cl

                @pl.when((g > 0) & (c - A_NBUF < NBLK))
                def _(c=c, b=b):
                    s_desc(c - A_NBUF, b).wait()

                finish_block(c, b)

                @pl.when(g < A_GROUPS - 1)
                def _(c=c, b=b):
                    start_block(c + A_NBUF, b)

            return 0

        lax.fori_loop(0, A_GROUPS, group, 0)

        for b in range(A_NBUF):
            c = c0 + A_PER_W - A_NBUF + b

            @pl.when(c < NBLK)
            def _(c=c, b=b):
                s_desc(c, b).wait()

    return format_kernel


def _make_lookup():
    mesh = plsc.VectorSubcoreMesh(core_axis_name="c", subcore_axis_name="s")

    @functools.partial(
        pl.kernel,
        mesh=mesh,
        out_type=jax.ShapeDtypeStruct((HIST, HT, BC, 8, BLK), jnp.float32),
        scratch_types=[
            pltpu.VMEM((U_PER_W * BLK,), jnp.int32),
            *[pltpu.VMEM((BLK, HIDDEN), jnp.float32) for _ in range(B_NBUF)],
            *[pltpu.VMEM((HT, 8, BLK), jnp.float32) for _ in range(B_NBUF)],
            *[pltpu.SemaphoreType.DMA for _ in range(2 * B_NBUF)],
        ],
        compiler_params=pltpu.CompilerParams(
            use_tc_tiling_on_sc=False, needs_layout_passes=False
        ),
    )
    def lookup_kernel(idx_hbm, table_hbm, out_hbm, idx_v, *bufs):
        rows = bufs[:B_NBUF]
        outt = bufs[B_NBUF : 2 * B_NBUF]
        sg = bufs[2 * B_NBUF : 3 * B_NBUF]
        ss = bufs[3 * B_NBUF : 4 * B_NBUF]
        wid = lax.axis_index("s") * NUM_CORES + lax.axis_index("c")
        u0 = wid * U_PER_W

        pltpu.sync_copy(idx_hbm.at[pl.ds(u0 * BLK, U_PER_W * BLK)], idx_v)

        lane = lax.iota(jnp.int32, 16)
        row_k = [lane + 16 * k for k in range(8)]

        def g_desc(ul, b):
            src = table_hbm.at[idx_v.at[pl.ds(ul * BLK, BLK)]]
            return pltpu.make_async_copy(src, rows[b], sg[b])

        def s_desc(ul, b):
            ug = u0 + ul
            j = ug // BC
            bc = ug % BC
            return pltpu.make_async_copy(outt[b], out_hbm.at[j, :, bc], ss[b])

        def transpose(b):
            @plsc.parallel_loop(0, HIDDEN, step=1, unroll=4)
            def h0body(h0):
                hh = (lane + h0) & 63
                trv = hh >> 3
                h8v = hh & 7
                for k in range(8):
                    vec = plsc.load_gather(rows[b], [row_k[k], hh])
                    plsc.store_scatter(outt[b], [trv, h8v, row_k[k]], vec)

        for b in range(B_NBUF):
            g_desc(b, b).start()

        def group(g, _):
            for b in range(B_NBUF):
                ul = g * B_NBUF + b

                @pl.when(g > 0)
                def _(ul=ul, b=b):
                    s_desc(ul - B_NBUF, b).wait()

                g_desc(ul, b).wait()
                transpose(b)
                s_desc(ul, b).start()

                @pl.when(g < B_GROUPS - 1)
                def _(ul=ul, b=b):
                    g_desc(ul + B_NBUF, b).start()

            return 0

        lax.fori_loop(0, B_GROUPS, group, 0)

        for b in range(B_NBUF):
            s_desc(U_PER_W - B_NBUF + b, b).wait()

    return lookup_kernel


_format = _make_format()
_lookup = _make_lookup()


@jax.jit
def kernel(tokens, embedding):
    idx = tokens.astype(jnp.int32).T.reshape(B_TOTAL)
    table = _format(embedding.T).reshape(VPAD, HIDDEN)
    out5 = _lookup(idx, table)
    return out5.transpose(2, 4, 0, 1, 3).reshape(BATCH, HIST, HIDDEN)
